# baseline (speedup 1.0000x reference)
import functools
import math

import jax
import jax.numpy as jnp
from jax import lax
from jax.experimental import pallas as pl
from jax.experimental.pallas import tpu as pltpu
from jax.experimental.pallas import tpu_sc as plsc

VOCAB = 1000000
D = 64
BATCH = 16384
HIST = 50
SCALE = math.sqrt(D)

NC = 2
NS = 16
NW = NC * NS

NSPLIT = 2
BSLICE = BATCH // NSPLIT
BPW = BSLICE // NW
NB = 8
STEPS = BPW // NB


def _sc_body(table_hbm, x_hbm, out_hbm,
             idx_v, ga, gb, sa, sb,
             gsem_a, gsem_b, ssem_a, ssem_b):
    wid = lax.axis_index("s") * NC + lax.axis_index("c")
    b0 = wid * BPW

    pltpu.sync_copy(x_hbm.at[pl.ds(b0, BPW)], idx_v)

    tail_mult = jnp.where(lax.iota(jnp.int32, 16) >= 14, 2, 1)

    @plsc.parallel_loop(0, BPW, unroll=4)
    def dbl(r):
        for j in range(HIST // 16):
            sl = pl.ds(16 * j, 16)
            idx_v[r, sl] = idx_v[r, sl] * 2
        tl = pl.ds(HIST - 16, 16)
        idx_v[r, tl] = idx_v[r, tl] * tail_mult

    gbufs = (ga, gb)
    sbufs = (sa, sb)
    gsems = (gsem_a, gsem_b)
    ssems = (ssem_a, ssem_b)

    def gather_start(s, b):
        for n in range(NB):
            pltpu.async_copy(
                table_hbm.at[idx_v.at[s * NB + n]], gbufs[b].at[n], gsems[b])

    def gather_wait(s, b):
        for n in range(NB):
            pltpu.make_async_copy(
                table_hbm.at[idx_v.at[s * NB + n]], gbufs[b].at[n],
                gsems[b]).wait()

    gather_start(0, 0)
    gather_start(1, 1)

    def pair(i, _):
        for b in range(2):
            s = 2 * i + b
            gather_wait(s, b)

            @pl.when(i >= 1)
            def _():
                pltpu.make_async_copy(
                    sbufs[b], out_hbm.at[pl.ds(b0 + (s - 2) * NB, NB)],
                    ssems[b]).wait()

            for n in range(NB):

                @plsc.parallel_loop(0, HIST, unroll=10)
                def row(r, n=n):
                    for j in range(D // 16):
                        sl = pl.ds(16 * j, 16)
                        sbufs[b][n, r, sl] = gbufs[b][n, r, sl] * SCALE

            pltpu.async_copy(
                sbufs[b], out_hbm.at[pl.ds(b0 + s * NB, NB)], ssems[b])

            @pl.when(i < STEPS // 2 - 1)
            def _():
                gather_start(s + 2, b)
        return _

    lax.fori_loop(0, STEPS // 2, pair, None)

    for b in range(2):
        s = STEPS - 2 + b
        pltpu.make_async_copy(
            sbufs[b], out_hbm.at[pl.ds(b0 + s * NB, NB)], ssems[b]).wait()


@jax.jit
def kernel(x, table):
    mesh = plsc.VectorSubcoreMesh(
        core_axis_name="c", subcore_axis_name="s",
        num_cores=NC, num_subcores=NS)
    run = pl.kernel(
        _sc_body,
        out_type=jax.ShapeDtypeStruct((BSLICE, HIST, D), jnp.float32),
        mesh=mesh,
        scratch_types=[
            pltpu.VMEM((BPW, HIST), jnp.int32),
            pltpu.VMEM((NB, HIST, D), jnp.float32),
            pltpu.VMEM((NB, HIST, D), jnp.float32),
            pltpu.VMEM((NB, HIST, D), jnp.float32),
            pltpu.VMEM((NB, HIST, D), jnp.float32),
            pltpu.SemaphoreType.DMA,
            pltpu.SemaphoreType.DMA,
            pltpu.SemaphoreType.DMA,
            pltpu.SemaphoreType.DMA,
        ],
        compiler_params=pltpu.CompilerParams(use_tc_tiling_on_sc=False),
    )
    tp = jnp.pad(table, ((0, 0), (0, D))).reshape(2 * VOCAB, D)
    outs = [run(tp, lax.slice_in_dim(x, i * BSLICE, (i + 1) * BSLICE, axis=0))
            for i in range(NSPLIT)]
    return jnp.concatenate(outs, axis=0)


# speedup vs baseline: 1.0569x; 1.0569x over previous
import functools
import math

import jax
import jax.numpy as jnp
from jax import lax
from jax.experimental import pallas as pl
from jax.experimental.pallas import tpu as pltpu
from jax.experimental.pallas import tpu_sc as plsc

VOCAB = 1000000
D = 64
BATCH = 16384
HIST = 50
SCALE = math.sqrt(D)

NC = 2
NS = 16
NW = NC * NS

BPW = BATCH // NW
NB = 4
NGB = 4
STEPS = BPW // NB


def _sc_body(table_hbm, x_hbm, out_hbm,
             idx_v, ga, gb, gc, gd, sa, sb,
             gsem_a, gsem_b, gsem_c, gsem_d, ssem_a, ssem_b):
    wid = lax.axis_index("s") * NC + lax.axis_index("c")
    b0 = wid * BPW

    pltpu.sync_copy(x_hbm.at[pl.ds(b0, BPW)], idx_v)

    tail_mult = jnp.where(lax.iota(jnp.int32, 16) >= 14, 2, 1)

    @plsc.parallel_loop(0, BPW, unroll=4)
    def dbl(r):
        for j in range(HIST // 16):
            sl = pl.ds(16 * j, 16)
            idx_v[r, sl] = idx_v[r, sl] * 2
        tl = pl.ds(HIST - 16, 16)
        idx_v[r, tl] = idx_v[r, tl] * tail_mult

    gbufs = (ga, gb, gc, gd)
    sbufs = (sa, sb)
    gsems = (gsem_a, gsem_b, gsem_c, gsem_d)
    ssems = (ssem_a, ssem_b)

    def gather_start(s, b):
        for n in range(NB):
            pltpu.async_copy(
                table_hbm.at[idx_v.at[s * NB + n]], gbufs[b].at[n], gsems[b])

    def gather_wait(s, b):
        for n in range(NB):
            pltpu.make_async_copy(
                table_hbm.at[idx_v.at[s * NB + n]], gbufs[b].at[n],
                gsems[b]).wait()

    for b in range(NGB):
        gather_start(b, b)

    def quad(i, _):
        for b in range(NGB):
            s = NGB * i + b
            sb_ = b & 1
            gather_wait(s, b)

            @pl.when(s >= 2)
            def _():
                pltpu.make_async_copy(
                    sbufs[sb_], out_hbm.at[pl.ds(b0 + (s - 2) * NB, NB)],
                    ssems[sb_]).wait()

            for n in range(NB):

                @plsc.parallel_loop(0, HIST, unroll=5)
                def row(r, n=n):
                    for j in range(D // 16):
                        sl = pl.ds(16 * j, 16)
                        sbufs[sb_][n, r, sl] = gbufs[b][n, r, sl] * SCALE

            pltpu.async_copy(
                sbufs[sb_], out_hbm.at[pl.ds(b0 + s * NB, NB)], ssems[sb_])

            @pl.when(i < STEPS // NGB - 1)
            def _():
                gather_start(s + NGB, b)
        return _

    lax.fori_loop(0, STEPS // NGB, quad, None)

    for sb_ in range(2):
        s = STEPS - 2 + sb_
        pltpu.make_async_copy(
            sbufs[s & 1], out_hbm.at[pl.ds(b0 + s * NB, NB)],
            ssems[s & 1]).wait()


@jax.jit
def kernel(x, table):
    mesh = plsc.VectorSubcoreMesh(
        core_axis_name="c", subcore_axis_name="s",
        num_cores=NC, num_subcores=NS)
    run = pl.kernel(
        _sc_body,
        out_type=jax.ShapeDtypeStruct((BATCH, HIST, D), jnp.float32),
        mesh=mesh,
        scratch_types=[
            pltpu.VMEM((BPW, HIST), jnp.int32),
            pltpu.VMEM((NB, HIST, D), jnp.float32),
            pltpu.VMEM((NB, HIST, D), jnp.float32),
            pltpu.VMEM((NB, HIST, D), jnp.float32),
            pltpu.VMEM((NB, HIST, D), jnp.float32),
            pltpu.VMEM((NB, HIST, D), jnp.float32),
            pltpu.VMEM((NB, HIST, D), jnp.float32),
            pltpu.SemaphoreType.DMA,
            pltpu.SemaphoreType.DMA,
            pltpu.SemaphoreType.DMA,
            pltpu.SemaphoreType.DMA,
            pltpu.SemaphoreType.DMA,
            pltpu.SemaphoreType.DMA,
        ],
        compiler_params=pltpu.CompilerParams(use_tc_tiling_on_sc=False),
    )
    tp = jnp.pad(table, ((0, 0), (0, D))).reshape(2 * VOCAB, D)
    return run(tp, x)
